# Initial kernel scaffold; baseline (speedup 1.0000x reference)
import jax
import jax.numpy as jnp
from jax.experimental import pallas as pl


def kernel(standard_sample, edge_index, mean_param, diag_param, post_diag_param, alpha1, alpha2, gamma):
    raise NotImplementedError("write your pallas kernel here")


# speedup vs baseline: 7.6071x; 7.6071x over previous
import functools

import jax
import jax.numpy as jnp
from jax import lax
from jax.experimental import pallas as pl
from jax.experimental.pallas import tpu as pltpu
from jax.experimental.pallas import tpu_sc as plsc

N_NODES = 100000
N_SAMPLES = 10
N_EDGES = 1600000
W = 16
N_PAD = 100352
NC, NS = 2, 16
NW = NC * NS
CHUNK = 128
CPT = 391
E_PAD = NW * CPT * CHUNK
ROWS_PER_TILE = N_PAD // NS
ZROWS = 784
NB = 2048


def _prep_body(z_ref, dg_ref, pd_ref, mn_ref, o_ref):
    z = z_ref[...]
    std = jax.nn.softplus(dg_ref[...])
    x = std * z
    col = lax.broadcasted_iota(jnp.int32, (NB, W), 1)
    x = jnp.where(col == 10, 1.0, x)
    x = jnp.where(col == 11, jax.nn.softplus(pd_ref[...]), x)
    x = jnp.where(col == 12, mn_ref[...], x)
    o_ref[...] = x


def _final_body(s_ref, xs_ref, a0_ref, a1_ref, o_ref):
    alpha1 = s_ref[0, 0]
    alpha2 = s_ref[0, 1]
    gamma = s_ref[0, 2]
    xs = xs_ref[...]
    aggr = a0_ref[...] + a1_ref[...]
    deg = jnp.maximum(aggr[:, 10:11], 1.0)
    self_w = jnp.exp(gamma * jnp.log(deg))
    out = xs[:, 11:12] * (alpha1 * self_w * xs + alpha2 * aggr) + xs[:, 12:13]
    o_ref[...] = out


def _sc_body(xs_hbm, src_hbm, dst_hbm, zeros_hbm, out_hbm,
             src_v, dst_v, rows_v, zb_v, acc_sh, sem):
    c = lax.axis_index("c")
    s = lax.axis_index("s")
    wid = s * NC + c

    pltpu.sync_copy(zeros_hbm, zb_v)
    for r in range(ROWS_PER_TILE // ZROWS):
        pltpu.sync_copy(zb_v, acc_sh.at[pl.ds(s * ROWS_PER_TILE + r * ZROWS, ZROWS)])
    plsc.subcore_barrier()

    def body(i, carry):
        pltpu.sync_copy(src_hbm.at[wid, i], src_v)
        pltpu.sync_copy(dst_hbm.at[wid, i], dst_v)
        pltpu.async_copy(xs_hbm.at[src_v], rows_v, sem).wait()
        pltpu.sync_copy(rows_v, acc_sh.at[dst_v], add=True)
        return carry

    lax.fori_loop(0, CPT, body, 0)
    plsc.subcore_barrier()

    pltpu.sync_copy(acc_sh.at[pl.ds(s * ROWS_PER_TILE, ROWS_PER_TILE)],
                    out_hbm.at[c, pl.ds(s * ROWS_PER_TILE, ROWS_PER_TILE)])


_sc_call = functools.partial(
    pl.kernel,
    out_type=jax.ShapeDtypeStruct((NC, N_PAD, W), jnp.float32),
    mesh=plsc.VectorSubcoreMesh(core_axis_name="c", subcore_axis_name="s"),
    scratch_types=[
        pltpu.VMEM((CHUNK,), jnp.int32),
        pltpu.VMEM((CHUNK,), jnp.int32),
        pltpu.VMEM((CHUNK, W), jnp.float32),
        pltpu.VMEM((ZROWS, W), jnp.float32),
        pltpu.VMEM_SHARED((N_PAD, W), jnp.float32),
        pltpu.SemaphoreType.DMA,
    ],
    compiler_params=pltpu.CompilerParams(use_tc_tiling_on_sc=False),
)(_sc_body)


def kernel(standard_sample, edge_index, mean_param, diag_param, post_diag_param,
           alpha1, alpha2, gamma):
    f32 = jnp.float32
    z_t = jnp.pad(standard_sample.astype(f32).T,
                  ((0, N_PAD - N_NODES), (0, W - N_SAMPLES)))
    dg = jnp.pad(diag_param, (0, N_PAD - N_NODES)).reshape(N_PAD, 1)
    pd = jnp.pad(post_diag_param, (0, N_PAD - N_NODES)).reshape(N_PAD, 1)
    mn = jnp.pad(mean_param, (0, N_PAD - N_NODES)).reshape(N_PAD, 1)
    src = jnp.pad(edge_index[0], (0, E_PAD - N_EDGES)).reshape(NW, CPT, CHUNK)
    dst = jnp.pad(edge_index[1], (0, E_PAD - N_EDGES),
                  constant_values=N_NODES).reshape(NW, CPT, CHUNK)
    zeros_hbm = jnp.zeros((ZROWS, W), f32)
    scalars = jnp.stack([alpha1, alpha2, gamma]).astype(f32).reshape(1, 3)

    grid = (N_PAD // NB,)
    row_spec = pl.BlockSpec((NB, W), lambda i: (i, 0))
    col_spec = pl.BlockSpec((NB, 1), lambda i: (i, 0))

    xs_rows = pl.pallas_call(
        _prep_body,
        grid=grid,
        in_specs=[row_spec, col_spec, col_spec, col_spec],
        out_specs=row_spec,
        out_shape=jax.ShapeDtypeStruct((N_PAD, W), f32),
    )(z_t, dg, pd, mn)

    acc = _sc_call(xs_rows, src, dst, zeros_hbm)

    out_rows = pl.pallas_call(
        _final_body,
        grid=grid,
        in_specs=[pl.BlockSpec(memory_space=pltpu.SMEM),
                  row_spec, row_spec, row_spec],
        out_specs=row_spec,
        out_shape=jax.ShapeDtypeStruct((N_PAD, W), f32),
    )(scalars, xs_rows, acc[0], acc[1])

    return out_rows[:N_NODES, :N_SAMPLES].T


# speedup vs baseline: 9.3509x; 1.2292x over previous
import functools

import jax
import jax.numpy as jnp
from jax import lax
from jax.experimental import pallas as pl
from jax.experimental.pallas import tpu as pltpu
from jax.experimental.pallas import tpu_sc as plsc

N_NODES = 100000
N_SAMPLES = 10
N_EDGES = 1600000
W = 16
N_PAD = 100352
NC, NS = 2, 16
NW = NC * NS
CHUNK = 128
KC = 8
NBLK = 50
CPT = NBLK * KC
E_PAD = NW * CPT * CHUNK
ROWS_PER_TILE = N_PAD // NS
ZROWS = 784
NB = 2048


def _prep_body(z_ref, dg_ref, pd_ref, mn_ref, o_ref):
    z = z_ref[...]
    std = jax.nn.softplus(dg_ref[...])
    x = std * z
    col = lax.broadcasted_iota(jnp.int32, (NB, W), 1)
    x = jnp.where(col == 10, 1.0, x)
    x = jnp.where(col == 11, jax.nn.softplus(pd_ref[...]), x)
    x = jnp.where(col == 12, mn_ref[...], x)
    o_ref[...] = x


def _final_body(s_ref, xs_ref, a0_ref, a1_ref, o_ref):
    alpha1 = s_ref[0, 0]
    alpha2 = s_ref[0, 1]
    gamma = s_ref[0, 2]
    xs = xs_ref[...]
    aggr = a0_ref[...] + a1_ref[...]
    deg = jnp.maximum(aggr[:, 10:11], 1.0)
    self_w = jnp.exp(gamma * jnp.log(deg))
    out = xs[:, 11:12] * (alpha1 * self_w * xs + alpha2 * aggr) + xs[:, 12:13]
    o_ref[...] = out


def _sc_body(xs_hbm, edges_hbm, zeros_hbm, out_hbm,
             idx_v, rows_v, acc_sh, gsem, ssem):
    c = lax.axis_index("c")
    s = lax.axis_index("s")
    wid = s * NC + c

    for r in range(ROWS_PER_TILE // ZROWS):
        pltpu.sync_copy(zeros_hbm, acc_sh.at[pl.ds(s * ROWS_PER_TILE + r * ZROWS, ZROWS)])
    plsc.subcore_barrier()

    def gather(k):
        return pltpu.async_copy(xs_hbm.at[idx_v.at[k, 0]], rows_v.at[k], gsem)

    def scatter(k):
        return pltpu.async_copy(rows_v.at[k], acc_sh.at[idx_v.at[k, 1]], ssem,
                                add=True)

    def body(blk, carry):
        pltpu.sync_copy(edges_hbm.at[wid, blk], idx_v)
        gather(0)
        for k in range(1, KC):
            gather(k)
            pltpu.make_async_copy(xs_hbm.at[idx_v.at[k - 1, 0]],
                                  rows_v.at[k - 1], gsem).wait()
            scatter(k - 1)
        pltpu.make_async_copy(xs_hbm.at[idx_v.at[KC - 1, 0]],
                              rows_v.at[KC - 1], gsem).wait()
        scatter(KC - 1)
        for k in range(KC):
            pltpu.make_async_copy(rows_v.at[k], acc_sh.at[idx_v.at[k, 1]],
                                  ssem).wait()
        return carry

    lax.fori_loop(0, NBLK, body, 0)
    plsc.subcore_barrier()

    pltpu.sync_copy(acc_sh.at[pl.ds(s * ROWS_PER_TILE, ROWS_PER_TILE)],
                    out_hbm.at[c, pl.ds(s * ROWS_PER_TILE, ROWS_PER_TILE)])


_sc_call = functools.partial(
    pl.kernel,
    out_type=jax.ShapeDtypeStruct((NC, N_PAD, W), jnp.float32),
    mesh=plsc.VectorSubcoreMesh(core_axis_name="c", subcore_axis_name="s"),
    scratch_types=[
        pltpu.VMEM((KC, 2, CHUNK), jnp.int32),
        pltpu.VMEM((KC, CHUNK, W), jnp.float32),
        pltpu.VMEM_SHARED((N_PAD, W), jnp.float32),
        pltpu.SemaphoreType.DMA,
        pltpu.SemaphoreType.DMA,
    ],
    compiler_params=pltpu.CompilerParams(use_tc_tiling_on_sc=False),
)(_sc_body)


def kernel(standard_sample, edge_index, mean_param, diag_param, post_diag_param,
           alpha1, alpha2, gamma):
    f32 = jnp.float32
    z_t = jnp.pad(standard_sample.astype(f32).T,
                  ((0, N_PAD - N_NODES), (0, W - N_SAMPLES)))
    dg = jnp.pad(diag_param, (0, N_PAD - N_NODES)).reshape(N_PAD, 1)
    pd = jnp.pad(post_diag_param, (0, N_PAD - N_NODES)).reshape(N_PAD, 1)
    mn = jnp.pad(mean_param, (0, N_PAD - N_NODES)).reshape(N_PAD, 1)
    src = jnp.pad(edge_index[0], (0, E_PAD - N_EDGES)
                  ).reshape(NW, NBLK, KC, 1, CHUNK)
    dst = jnp.pad(edge_index[1], (0, E_PAD - N_EDGES),
                  constant_values=N_NODES).reshape(NW, NBLK, KC, 1, CHUNK)
    edges = jnp.concatenate([src, dst], axis=3)
    zeros_hbm = jnp.zeros((ZROWS, W), f32)
    scalars = jnp.stack([alpha1, alpha2, gamma]).astype(f32).reshape(1, 3)

    grid = (N_PAD // NB,)
    row_spec = pl.BlockSpec((NB, W), lambda i: (i, 0))
    col_spec = pl.BlockSpec((NB, 1), lambda i: (i, 0))

    xs_rows = pl.pallas_call(
        _prep_body,
        grid=grid,
        in_specs=[row_spec, col_spec, col_spec, col_spec],
        out_specs=row_spec,
        out_shape=jax.ShapeDtypeStruct((N_PAD, W), f32),
    )(z_t, dg, pd, mn)

    acc = _sc_call(xs_rows, edges, zeros_hbm)

    out_rows = pl.pallas_call(
        _final_body,
        grid=grid,
        in_specs=[pl.BlockSpec(memory_space=pltpu.SMEM),
                  row_spec, row_spec, row_spec],
        out_specs=row_spec,
        out_shape=jax.ShapeDtypeStruct((N_PAD, W), f32),
    )(scalars, xs_rows, acc[0], acc[1])

    return out_rows[:N_NODES, :N_SAMPLES].T


# speedup vs baseline: 11.5679x; 1.2371x over previous
import functools

import jax
import jax.numpy as jnp
from jax import lax
from jax.experimental import pallas as pl
from jax.experimental.pallas import tpu as pltpu
from jax.experimental.pallas import tpu_sc as plsc

N_NODES = 100000
N_SAMPLES = 10
N_EDGES = 1600000
W = 16
N_PAD = 100352
NC, NS = 2, 16
NW = NC * NS
CHUNK = 128
KC = 8
NBLK = 50
CPT = NBLK * KC
E_PAD = NW * CPT * CHUNK
ROWS_PER_TILE = N_PAD // NS
ZROWS = 784
NBC = 2048


def _prep_body(z_ref, dg_ref, pd_ref, mn_ref, o_ref):
    z = z_ref[...]
    std = jax.nn.softplus(dg_ref[...])
    x = std * z
    row = lax.broadcasted_iota(jnp.int32, (W, NBC), 0)
    x = jnp.where(row == 10, 1.0, x)
    x = jnp.where(row == 11, jax.nn.softplus(pd_ref[...]), x)
    x = jnp.where(row == 12, mn_ref[...], x)
    o_ref[...] = x


def _final_body(s_ref, xs_ref, a0_ref, a1_ref, o_ref):
    alpha1 = s_ref[0, 0]
    alpha2 = s_ref[0, 1]
    gamma = s_ref[0, 2]
    xs = xs_ref[...]
    aggr = a0_ref[...] + a1_ref[...]
    deg = jnp.maximum(aggr[10:11, :], 1.0)
    self_w = jnp.exp(gamma * jnp.log(deg))
    out = xs[11:12, :] * (alpha1 * self_w * xs + alpha2 * aggr) + xs[12:13, :]
    o_ref[...] = out


def _sc_body(xs_hbm, src_hbm, dst_hbm, zeros_hbm, out_hbm,
             src_v, dst_v, rows_v, acc_sh, gsem, ssem):
    c = lax.axis_index("c")
    s = lax.axis_index("s")
    wid = s * NC + c

    for r in range(ROWS_PER_TILE // ZROWS):
        pltpu.sync_copy(zeros_hbm,
                        acc_sh.at[pl.ds(s * ROWS_PER_TILE + r * ZROWS, ZROWS)])
    plsc.subcore_barrier()

    def gather(k):
        return pltpu.async_copy(xs_hbm.at[src_v.at[k]], rows_v.at[k], gsem)

    def scatter(k):
        return pltpu.async_copy(rows_v.at[k], acc_sh.at[dst_v.at[k]], ssem,
                                add=True)

    def body(blk, carry):
        pltpu.sync_copy(src_hbm.at[wid, blk], src_v)
        pltpu.sync_copy(dst_hbm.at[wid, blk], dst_v)
        gather(0)
        for k in range(1, KC):
            gather(k)
            pltpu.make_async_copy(xs_hbm.at[src_v.at[k - 1]],
                                  rows_v.at[k - 1], gsem).wait()
            scatter(k - 1)
        pltpu.make_async_copy(xs_hbm.at[src_v.at[KC - 1]],
                              rows_v.at[KC - 1], gsem).wait()
        scatter(KC - 1)
        for k in range(KC):
            pltpu.make_async_copy(rows_v.at[k], acc_sh.at[dst_v.at[k]],
                                  ssem).wait()
        return carry

    lax.fori_loop(0, NBLK, body, 0)
    plsc.subcore_barrier()

    pltpu.sync_copy(acc_sh.at[pl.ds(s * ROWS_PER_TILE, ROWS_PER_TILE)],
                    out_hbm.at[c, pl.ds(s * ROWS_PER_TILE, ROWS_PER_TILE)])


_sc_call = functools.partial(
    pl.kernel,
    out_type=jax.ShapeDtypeStruct((NC, N_PAD, W), jnp.float32),
    mesh=plsc.VectorSubcoreMesh(core_axis_name="c", subcore_axis_name="s"),
    scratch_types=[
        pltpu.VMEM((KC, CHUNK), jnp.int32),
        pltpu.VMEM((KC, CHUNK), jnp.int32),
        pltpu.VMEM((KC, CHUNK, W), jnp.float32),
        pltpu.VMEM_SHARED((N_PAD, W), jnp.float32),
        pltpu.SemaphoreType.DMA,
        pltpu.SemaphoreType.DMA,
    ],
    compiler_params=pltpu.CompilerParams(use_tc_tiling_on_sc=False),
)(_sc_body)


def kernel(standard_sample, edge_index, mean_param, diag_param, post_diag_param,
           alpha1, alpha2, gamma):
    f32 = jnp.float32
    zq = jnp.pad(standard_sample.astype(f32),
                 ((0, W - N_SAMPLES), (0, N_PAD - N_NODES)))
    dg = jnp.pad(diag_param, (0, N_PAD - N_NODES)).reshape(1, N_PAD)
    pd = jnp.pad(post_diag_param, (0, N_PAD - N_NODES)).reshape(1, N_PAD)
    mn = jnp.pad(mean_param, (0, N_PAD - N_NODES)).reshape(1, N_PAD)
    src = jnp.pad(edge_index[0], (0, E_PAD - N_EDGES)
                  ).reshape(NW, NBLK, KC, CHUNK)
    dst = jnp.pad(edge_index[1], (0, E_PAD - N_EDGES),
                  constant_values=N_NODES).reshape(NW, NBLK, KC, CHUNK)
    zeros_hbm = jnp.zeros((ZROWS, W), f32)
    scalars = jnp.stack([alpha1, alpha2, gamma]).astype(f32).reshape(1, 3)

    grid = (N_PAD // NBC,)
    blk_spec = pl.BlockSpec((W, NBC), lambda i: (0, i))
    par_spec = pl.BlockSpec((1, NBC), lambda i: (0, i))

    xs_packed = pl.pallas_call(
        _prep_body,
        grid=grid,
        in_specs=[blk_spec, par_spec, par_spec, par_spec],
        out_specs=blk_spec,
        out_shape=jax.ShapeDtypeStruct((W, N_PAD), f32),
    )(zq, dg, pd, mn)
    xs_rows = xs_packed.T

    acc = _sc_call(xs_rows, src, dst, zeros_hbm)

    out_packed = pl.pallas_call(
        _final_body,
        grid=grid,
        in_specs=[pl.BlockSpec(memory_space=pltpu.SMEM),
                  blk_spec, blk_spec, blk_spec],
        out_specs=blk_spec,
        out_shape=jax.ShapeDtypeStruct((W, N_PAD), f32),
    )(scalars, xs_packed, acc[0].T, acc[1].T)

    return out_packed[:N_SAMPLES, :N_NODES]


# speedup vs baseline: 12.3121x; 1.0643x over previous
import functools

import jax
import jax.numpy as jnp
from jax import lax
from jax.experimental import pallas as pl
from jax.experimental.pallas import tpu as pltpu
from jax.experimental.pallas import tpu_sc as plsc

N_NODES = 100000
N_SAMPLES = 10
N_EDGES = 1600000
W = 16
N_PAD = 100352
NC, NS = 2, 16
NW = NC * NS
CHUNK = 128
KC = 8
NBLK0 = 59
NBLK1 = 41
TOT_BLK = NS * (NBLK0 + NBLK1)
E_PAD = TOT_BLK * KC * CHUNK
ROWS_PER_TILE = N_PAD // NS
ZROWS = 784
NBC = 2048


def _prep_body(z_ref, dg_ref, pd_ref, mn_ref, o_ref):
    z = z_ref[...]
    std = jax.nn.softplus(dg_ref[...])
    x = std * z
    row = lax.broadcasted_iota(jnp.int32, (W, NBC), 0)
    x = jnp.where(row == 10, 1.0, x)
    x = jnp.where(row == 11, jax.nn.softplus(pd_ref[...]), x)
    x = jnp.where(row == 12, mn_ref[...], x)
    o_ref[...] = x


def _final_body(s_ref, xs_ref, a0_ref, a1_ref, o_ref):
    alpha1 = s_ref[0, 0]
    alpha2 = s_ref[0, 1]
    gamma = s_ref[0, 2]
    xs = xs_ref[...]
    aggr = a0_ref[...] + a1_ref[...]
    deg = jnp.maximum(aggr[10:11, :], 1.0)
    self_w = jnp.exp(gamma * jnp.log(deg))
    out = xs[11:12, :] * (alpha1 * self_w * xs + alpha2 * aggr) + xs[12:13, :]
    o_ref[...] = out


def _sc_body(xs_hbm, src_hbm, dst_hbm, zeros_hbm, out_hbm,
             src_v, dst_v, rows_v, acc_sh, gsem, ssem):
    c = lax.axis_index("c")
    s = lax.axis_index("s")
    base = jnp.where(c == 0, s * NBLK0, NS * NBLK0 + s * NBLK1)
    nblk = jnp.where(c == 0, NBLK0, NBLK1)

    for r in range(ROWS_PER_TILE // ZROWS):
        pltpu.sync_copy(zeros_hbm,
                        acc_sh.at[pl.ds(s * ROWS_PER_TILE + r * ZROWS, ZROWS)])
    plsc.subcore_barrier()

    def gather(k):
        return pltpu.async_copy(xs_hbm.at[src_v.at[k]], rows_v.at[k], gsem)

    def scatter(k):
        return pltpu.async_copy(rows_v.at[k], acc_sh.at[dst_v.at[k]], ssem,
                                add=True)

    def body(blk, carry):
        pltpu.sync_copy(src_hbm.at[base + blk], src_v)
        pltpu.sync_copy(dst_hbm.at[base + blk], dst_v)
        gather(0)
        for k in range(1, KC):
            gather(k)
            pltpu.make_async_copy(xs_hbm.at[src_v.at[k - 1]],
                                  rows_v.at[k - 1], gsem).wait()
            scatter(k - 1)
        pltpu.make_async_copy(xs_hbm.at[src_v.at[KC - 1]],
                              rows_v.at[KC - 1], gsem).wait()
        scatter(KC - 1)
        for k in range(KC):
            pltpu.make_async_copy(rows_v.at[k], acc_sh.at[dst_v.at[k]],
                                  ssem).wait()
        return carry

    lax.fori_loop(0, nblk, body, 0)
    plsc.subcore_barrier()

    pltpu.sync_copy(acc_sh.at[pl.ds(s * ROWS_PER_TILE, ROWS_PER_TILE)],
                    out_hbm.at[c, pl.ds(s * ROWS_PER_TILE, ROWS_PER_TILE)])


_sc_call = functools.partial(
    pl.kernel,
    out_type=jax.ShapeDtypeStruct((NC, N_PAD, W), jnp.float32),
    mesh=plsc.VectorSubcoreMesh(core_axis_name="c", subcore_axis_name="s"),
    scratch_types=[
        pltpu.VMEM((KC, CHUNK), jnp.int32),
        pltpu.VMEM((KC, CHUNK), jnp.int32),
        pltpu.VMEM((KC, CHUNK, W), jnp.float32),
        pltpu.VMEM_SHARED((N_PAD, W), jnp.float32),
        pltpu.SemaphoreType.DMA,
        pltpu.SemaphoreType.DMA,
    ],
    compiler_params=pltpu.CompilerParams(use_tc_tiling_on_sc=False),
)(_sc_body)


def kernel(standard_sample, edge_index, mean_param, diag_param, post_diag_param,
           alpha1, alpha2, gamma):
    f32 = jnp.float32
    zq = jnp.pad(standard_sample.astype(f32),
                 ((0, W - N_SAMPLES), (0, N_PAD - N_NODES)))
    dg = jnp.pad(diag_param, (0, N_PAD - N_NODES)).reshape(1, N_PAD)
    pd = jnp.pad(post_diag_param, (0, N_PAD - N_NODES)).reshape(1, N_PAD)
    mn = jnp.pad(mean_param, (0, N_PAD - N_NODES)).reshape(1, N_PAD)
    src = jnp.pad(edge_index[0], (0, E_PAD - N_EDGES)
                  ).reshape(TOT_BLK, KC, CHUNK)
    dst = jnp.pad(edge_index[1], (0, E_PAD - N_EDGES),
                  constant_values=N_NODES).reshape(TOT_BLK, KC, CHUNK)
    zeros_hbm = jnp.zeros((ZROWS, W), f32)
    scalars = jnp.stack([alpha1, alpha2, gamma]).astype(f32).reshape(1, 3)

    grid = (N_PAD // NBC,)
    blk_spec = pl.BlockSpec((W, NBC), lambda i: (0, i))
    par_spec = pl.BlockSpec((1, NBC), lambda i: (0, i))

    xs_packed = pl.pallas_call(
        _prep_body,
        grid=grid,
        in_specs=[blk_spec, par_spec, par_spec, par_spec],
        out_specs=blk_spec,
        out_shape=jax.ShapeDtypeStruct((W, N_PAD), f32),
    )(zq, dg, pd, mn)
    xs_rows = xs_packed.T

    acc = _sc_call(xs_rows, src, dst, zeros_hbm)

    out_packed = pl.pallas_call(
        _final_body,
        grid=grid,
        in_specs=[pl.BlockSpec(memory_space=pltpu.SMEM),
                  blk_spec, blk_spec, blk_spec],
        out_specs=blk_spec,
        out_shape=jax.ShapeDtypeStruct((W, N_PAD), f32),
    )(scalars, xs_packed, acc[0].T, acc[1].T)

    return out_packed[:N_SAMPLES, :N_NODES]


# speedup vs baseline: 14.5542x; 1.1821x over previous
import functools

import jax
import jax.numpy as jnp
from jax import lax
from jax.experimental import pallas as pl
from jax.experimental.pallas import tpu as pltpu
from jax.experimental.pallas import tpu_sc as plsc

N_NODES = 100000
N_SAMPLES = 10
N_EDGES = 1600000
W = 16
N_PAD = 100352
NC, NS = 2, 16
NW = NC * NS
CHUNK = 128
KC = 8
NBLK0 = 59
NBLK1 = 41
TOT_BLK = NS * (NBLK0 + NBLK1)
E_PAD = TOT_BLK * KC * CHUNK
ROWS_PER_TILE = N_PAD // NS
ZROWS = 784
NBC = 2048


def _prep_body(z_ref, dg_ref, pd_ref, mn_ref, o_ref):
    z = z_ref[...]
    std = jax.nn.softplus(dg_ref[...])
    x = std * z
    row = lax.broadcasted_iota(jnp.int32, (W, NBC), 0)
    x = jnp.where(row == 10, 1.0, x)
    x = jnp.where(row == 11, jax.nn.softplus(pd_ref[...]), x)
    x = jnp.where(row == 12, mn_ref[...], x)
    o_ref[...] = x


def _final_body(s_ref, xs_ref, a_ref, o_ref):
    alpha1 = s_ref[0, 0]
    alpha2 = s_ref[0, 1]
    gamma = s_ref[0, 2]
    xs = xs_ref[...]
    a = a_ref[...]
    aggr = (a[0] + a[1]).T
    deg = jnp.maximum(aggr[10:11, :], 1.0)
    self_w = jnp.exp(gamma * jnp.log(deg))
    out = xs[11:12, :] * (alpha1 * self_w * xs + alpha2 * aggr) + xs[12:13, :]
    o_ref[...] = out


def _sc_body(xs_hbm, src_hbm, dst_hbm, zeros_hbm, out_hbm,
             src_v, dst_v, rows_v, acc_sh, gsem, ssem):
    c = lax.axis_index("c")
    s = lax.axis_index("s")
    base = jnp.where(c == 0, s * NBLK0, NS * NBLK0 + s * NBLK1)
    nblk = jnp.where(c == 0, NBLK0, NBLK1)

    for r in range(ROWS_PER_TILE // ZROWS):
        pltpu.sync_copy(zeros_hbm,
                        acc_sh.at[pl.ds(s * ROWS_PER_TILE + r * ZROWS, ZROWS)])
    plsc.subcore_barrier()

    def gather(k):
        return pltpu.async_copy(xs_hbm.at[src_v.at[k]], rows_v.at[k], gsem)

    def scatter(k):
        return pltpu.async_copy(rows_v.at[k], acc_sh.at[dst_v.at[k]], ssem,
                                add=True)

    def body(blk, carry):
        pltpu.sync_copy(src_hbm.at[base + blk], src_v)
        pltpu.sync_copy(dst_hbm.at[base + blk], dst_v)
        gather(0)
        for k in range(1, KC):
            gather(k)
            pltpu.make_async_copy(xs_hbm.at[src_v.at[k - 1]],
                                  rows_v.at[k - 1], gsem).wait()
            scatter(k - 1)
        pltpu.make_async_copy(xs_hbm.at[src_v.at[KC - 1]],
                              rows_v.at[KC - 1], gsem).wait()
        scatter(KC - 1)
        for k in range(KC):
            pltpu.make_async_copy(rows_v.at[k], acc_sh.at[dst_v.at[k]],
                                  ssem).wait()
        return carry

    lax.fori_loop(0, nblk, body, 0)
    plsc.subcore_barrier()

    pltpu.sync_copy(acc_sh.at[pl.ds(s * ROWS_PER_TILE, ROWS_PER_TILE)],
                    out_hbm.at[c, pl.ds(s * ROWS_PER_TILE, ROWS_PER_TILE)])


_sc_call = functools.partial(
    pl.kernel,
    out_type=jax.ShapeDtypeStruct((NC, N_PAD, W), jnp.float32),
    mesh=plsc.VectorSubcoreMesh(core_axis_name="c", subcore_axis_name="s"),
    scratch_types=[
        pltpu.VMEM((KC, CHUNK), jnp.int32),
        pltpu.VMEM((KC, CHUNK), jnp.int32),
        pltpu.VMEM((KC, CHUNK, W), jnp.float32),
        pltpu.VMEM_SHARED((N_PAD, W), jnp.float32),
        pltpu.SemaphoreType.DMA,
        pltpu.SemaphoreType.DMA,
    ],
    compiler_params=pltpu.CompilerParams(use_tc_tiling_on_sc=False),
)(_sc_body)


def kernel(standard_sample, edge_index, mean_param, diag_param, post_diag_param,
           alpha1, alpha2, gamma):
    f32 = jnp.float32
    zq = jnp.pad(standard_sample.astype(f32),
                 ((0, W - N_SAMPLES), (0, N_PAD - N_NODES)))
    dg = jnp.pad(diag_param, (0, N_PAD - N_NODES)).reshape(1, N_PAD)
    pd = jnp.pad(post_diag_param, (0, N_PAD - N_NODES)).reshape(1, N_PAD)
    mn = jnp.pad(mean_param, (0, N_PAD - N_NODES)).reshape(1, N_PAD)
    src = jnp.pad(edge_index[0], (0, E_PAD - N_EDGES)
                  ).reshape(TOT_BLK, KC, CHUNK)
    dst = jnp.pad(edge_index[1], (0, E_PAD - N_EDGES),
                  constant_values=N_NODES).reshape(TOT_BLK, KC, CHUNK)
    zeros_hbm = jnp.zeros((ZROWS, W), f32)
    scalars = jnp.stack([alpha1, alpha2, gamma]).astype(f32).reshape(1, 3)

    grid = (N_PAD // NBC,)
    blk_spec = pl.BlockSpec((W, NBC), lambda i: (0, i))
    par_spec = pl.BlockSpec((1, NBC), lambda i: (0, i))

    xs_packed = pl.pallas_call(
        _prep_body,
        grid=grid,
        in_specs=[blk_spec, par_spec, par_spec, par_spec],
        out_specs=blk_spec,
        out_shape=jax.ShapeDtypeStruct((W, N_PAD), f32),
    )(zq, dg, pd, mn)
    xs_rows = xs_packed.T

    acc = _sc_call(xs_rows, src, dst, zeros_hbm)

    out_packed = pl.pallas_call(
        _final_body,
        grid=grid,
        in_specs=[pl.BlockSpec(memory_space=pltpu.SMEM),
                  blk_spec,
                  pl.BlockSpec((NC, NBC, W), lambda i: (0, i, 0))],
        out_specs=blk_spec,
        out_shape=jax.ShapeDtypeStruct((W, N_PAD), f32),
    )(scalars, xs_packed, acc)

    return out_packed[:N_SAMPLES, :N_NODES]


# speedup vs baseline: 14.8985x; 1.0237x over previous
import functools

import jax
import jax.numpy as jnp
from jax import lax
from jax.experimental import pallas as pl
from jax.experimental.pallas import tpu as pltpu
from jax.experimental.pallas import tpu_sc as plsc

N_NODES = 100000
N_SAMPLES = 10
N_EDGES = 1600000
W = 16
N_PAD = 100352
NC, NS = 2, 16
NW = NC * NS
CHUNK = 128
KC = 4
N_CHUNKS = N_EDGES // CHUNK
ROWS_PER_TILE = N_PAD // NS
ZROWS = 784
NBC = 2048


def _prep_body(z_ref, dg_ref, pd_ref, mn_ref, o_ref):
    z = z_ref[...]
    std = jax.nn.softplus(dg_ref[...])
    x = std * z
    row = lax.broadcasted_iota(jnp.int32, (W, NBC), 0)
    x = jnp.where(row == 10, 1.0, x)
    x = jnp.where(row == 11, jax.nn.softplus(pd_ref[...]), x)
    x = jnp.where(row == 12, mn_ref[...], x)
    o_ref[...] = x


def _final_body(s_ref, xs_ref, a_ref, o_ref):
    alpha1 = s_ref[0, 0]
    alpha2 = s_ref[0, 1]
    gamma = s_ref[0, 2]
    xs = xs_ref[...]
    a = a_ref[...]
    aggr = (a[0] + a[1]).T
    deg = jnp.maximum(aggr[10:11, :], 1.0)
    self_w = jnp.exp(gamma * jnp.log(deg))
    out = xs[11:12, :] * (alpha1 * self_w * xs + alpha2 * aggr) + xs[12:13, :]
    o_ref[...] = out


def _sc_body(xs_hbm, ei_hbm, zeros_hbm, out_hbm,
             src_v, dst_v, rows_v, acc_sh, gsem, ssem, dsem):
    c = lax.axis_index("c")
    s = lax.axis_index("s")
    base_chunk = jnp.where(
        c == 0,
        jnp.where(s < 5, s * 464, 5 * 464 + (s - 5) * 460),
        7380 + s * 320)
    nblk = jnp.where(c == 0, jnp.where(s < 5, 116, 115), 80)

    for r in range(ROWS_PER_TILE // ZROWS):
        pltpu.sync_copy(zeros_hbm,
                        acc_sh.at[pl.ds(s * ROWS_PER_TILE + r * ZROWS, ZROWS)])
    plsc.subcore_barrier()

    def gather(k):
        return pltpu.async_copy(xs_hbm.at[src_v.at[pl.ds(k * CHUNK, CHUNK)]],
                                rows_v.at[k], gsem)

    def scatter(k):
        return pltpu.async_copy(rows_v.at[k], acc_sh.at[dst_v.at[k]], ssem,
                                add=True)

    def body(blk, carry):
        e0 = (base_chunk + blk * KC) * CHUNK
        pltpu.sync_copy(ei_hbm.at[0, pl.ds(e0, KC * CHUNK)], src_v)
        for k in range(KC):
            pltpu.async_copy(ei_hbm.at[1, pl.ds(e0 + k * CHUNK, CHUNK)],
                             dst_v.at[k], dsem)
        for k in range(KC):
            pltpu.make_async_copy(ei_hbm.at[1, pl.ds(e0 + k * CHUNK, CHUNK)],
                                  dst_v.at[k], dsem).wait()
        gather(0)
        for k in range(1, KC):
            gather(k)
            pltpu.make_async_copy(xs_hbm.at[src_v.at[pl.ds((k - 1) * CHUNK,
                                                           CHUNK)]],
                                  rows_v.at[k - 1], gsem).wait()
            scatter(k - 1)
        pltpu.make_async_copy(xs_hbm.at[src_v.at[pl.ds((KC - 1) * CHUNK,
                                                       CHUNK)]],
                              rows_v.at[KC - 1], gsem).wait()
        scatter(KC - 1)
        for k in range(KC):
            pltpu.make_async_copy(rows_v.at[k], acc_sh.at[dst_v.at[k]],
                                  ssem).wait()
        return carry

    lax.fori_loop(0, nblk, body, 0)
    plsc.subcore_barrier()

    pltpu.sync_copy(acc_sh.at[pl.ds(s * ROWS_PER_TILE, ROWS_PER_TILE)],
                    out_hbm.at[c, pl.ds(s * ROWS_PER_TILE, ROWS_PER_TILE)])


_sc_call = functools.partial(
    pl.kernel,
    out_type=jax.ShapeDtypeStruct((NC, N_PAD, W), jnp.float32),
    mesh=plsc.VectorSubcoreMesh(core_axis_name="c", subcore_axis_name="s"),
    scratch_types=[
        pltpu.VMEM((KC * CHUNK,), jnp.int32),
        pltpu.VMEM((KC, CHUNK), jnp.int32),
        pltpu.VMEM((KC, CHUNK, W), jnp.float32),
        pltpu.VMEM_SHARED((N_PAD, W), jnp.float32),
        pltpu.SemaphoreType.DMA,
        pltpu.SemaphoreType.DMA,
        pltpu.SemaphoreType.DMA,
    ],
    compiler_params=pltpu.CompilerParams(use_tc_tiling_on_sc=False),
)(_sc_body)


def kernel(standard_sample, edge_index, mean_param, diag_param, post_diag_param,
           alpha1, alpha2, gamma):
    f32 = jnp.float32
    zq = jnp.pad(standard_sample.astype(f32),
                 ((0, W - N_SAMPLES), (0, N_PAD - N_NODES)))
    dg = jnp.pad(diag_param, (0, N_PAD - N_NODES)).reshape(1, N_PAD)
    pd = jnp.pad(post_diag_param, (0, N_PAD - N_NODES)).reshape(1, N_PAD)
    mn = jnp.pad(mean_param, (0, N_PAD - N_NODES)).reshape(1, N_PAD)
    zeros_hbm = jnp.zeros((ZROWS, W), f32)
    scalars = jnp.stack([alpha1, alpha2, gamma]).astype(f32).reshape(1, 3)

    grid = (N_PAD // NBC,)
    blk_spec = pl.BlockSpec((W, NBC), lambda i: (0, i))
    par_spec = pl.BlockSpec((1, NBC), lambda i: (0, i))

    xs_packed = pl.pallas_call(
        _prep_body,
        grid=grid,
        in_specs=[blk_spec, par_spec, par_spec, par_spec],
        out_specs=blk_spec,
        out_shape=jax.ShapeDtypeStruct((W, N_PAD), f32),
    )(zq, dg, pd, mn)
    xs_rows = xs_packed.T

    acc = _sc_call(xs_rows, edge_index, zeros_hbm)

    out_packed = pl.pallas_call(
        _final_body,
        grid=grid,
        in_specs=[pl.BlockSpec(memory_space=pltpu.SMEM),
                  blk_spec,
                  pl.BlockSpec((NC, NBC, W), lambda i: (0, i, 0))],
        out_specs=blk_spec,
        out_shape=jax.ShapeDtypeStruct((W, N_PAD), f32),
    )(scalars, xs_packed, acc)

    return out_packed[:N_SAMPLES, :N_NODES]


# speedup vs baseline: 16.3070x; 1.0945x over previous
import functools

import jax
import jax.numpy as jnp
from jax import lax
from jax.experimental import pallas as pl
from jax.experimental.pallas import tpu as pltpu
from jax.experimental.pallas import tpu_sc as plsc

N_NODES = 100000
N_SAMPLES = 10
N_EDGES = 1600000
W = 16
N_PAD = 100352
NC, NS = 2, 16
NW = NC * NS
CHUNK = 128
KC = 4
N_CHUNKS = N_EDGES // CHUNK
ROWS_PER_TILE = N_PAD // NS
ZROWS = 784
NBC = 2048


def _prep_body(z_ref, dg_ref, pd_ref, mn_ref, o_ref):
    z = z_ref[...]
    std = jax.nn.softplus(dg_ref[...])
    x = std * z
    row = lax.broadcasted_iota(jnp.int32, (W, NBC), 0)
    x = jnp.where(row == 10, 1.0, x)
    x = jnp.where(row == 11, jax.nn.softplus(pd_ref[...]), x)
    x = jnp.where(row == 12, mn_ref[...], x)
    o_ref[...] = x


def _final_body(s_ref, xs_ref, a_ref, o_ref):
    alpha1 = s_ref[0, 0]
    alpha2 = s_ref[0, 1]
    gamma = s_ref[0, 2]
    xs = xs_ref[...]
    a = a_ref[...]
    aggr = (a[0] + a[1]).T
    deg = jnp.maximum(aggr[10:11, :], 1.0)
    self_w = jnp.exp(gamma * jnp.log(deg))
    out = xs[11:12, :] * (alpha1 * self_w * xs + alpha2 * aggr) + xs[12:13, :]
    o_ref[...] = out


def _sc_body(xs_hbm, ei_hbm, zeros_hbm, out_hbm,
             src_v, dst_v, rows_v, acc_sh, gsem, ssem, dsem):
    c = lax.axis_index("c")
    s = lax.axis_index("s")
    base_blk = jnp.where(
        c == 0,
        jnp.where(s < 10, s * 99, 990 + (s - 10) * 98),
        1578 + jnp.where(s < 11, s * 97, 1067 + (s - 11) * 96))
    base_chunk = base_blk * KC
    nblk = jnp.where(c == 0, jnp.where(s < 10, 99, 98),
                     jnp.where(s < 11, 97, 96))

    for r in range(ROWS_PER_TILE // ZROWS):
        pltpu.sync_copy(zeros_hbm,
                        acc_sh.at[pl.ds(s * ROWS_PER_TILE + r * ZROWS, ZROWS)])
    plsc.subcore_barrier()

    def gather(k):
        return pltpu.async_copy(xs_hbm.at[src_v.at[pl.ds(k * CHUNK, CHUNK)]],
                                rows_v.at[k], gsem)

    def scatter(k):
        return pltpu.async_copy(rows_v.at[k], acc_sh.at[dst_v.at[k]], ssem,
                                add=True)

    def body(blk, carry):
        e0 = (base_chunk + blk * KC) * CHUNK
        pltpu.sync_copy(ei_hbm.at[0, pl.ds(e0, KC * CHUNK)], src_v)
        for k in range(KC):
            pltpu.async_copy(ei_hbm.at[1, pl.ds(e0 + k * CHUNK, CHUNK)],
                             dst_v.at[k], dsem)
        for k in range(KC):
            pltpu.make_async_copy(ei_hbm.at[1, pl.ds(e0 + k * CHUNK, CHUNK)],
                                  dst_v.at[k], dsem).wait()
        gather(0)
        for k in range(1, KC):
            gather(k)
            pltpu.make_async_copy(xs_hbm.at[src_v.at[pl.ds((k - 1) * CHUNK,
                                                           CHUNK)]],
                                  rows_v.at[k - 1], gsem).wait()
            scatter(k - 1)
        pltpu.make_async_copy(xs_hbm.at[src_v.at[pl.ds((KC - 1) * CHUNK,
                                                       CHUNK)]],
                              rows_v.at[KC - 1], gsem).wait()
        scatter(KC - 1)
        for k in range(KC):
            pltpu.make_async_copy(rows_v.at[k], acc_sh.at[dst_v.at[k]],
                                  ssem).wait()
        return carry

    lax.fori_loop(0, nblk, body, 0)
    plsc.subcore_barrier()

    pltpu.sync_copy(acc_sh.at[pl.ds(s * ROWS_PER_TILE, ROWS_PER_TILE)],
                    out_hbm.at[c, pl.ds(s * ROWS_PER_TILE, ROWS_PER_TILE)])


_sc_call = functools.partial(
    pl.kernel,
    out_type=jax.ShapeDtypeStruct((NC, N_PAD, W), jnp.float32),
    mesh=plsc.VectorSubcoreMesh(core_axis_name="c", subcore_axis_name="s"),
    scratch_types=[
        pltpu.VMEM((KC * CHUNK,), jnp.int32),
        pltpu.VMEM((KC, CHUNK), jnp.int32),
        pltpu.VMEM((KC, CHUNK, W), jnp.float32),
        pltpu.VMEM_SHARED((N_PAD, W), jnp.float32),
        pltpu.SemaphoreType.DMA,
        pltpu.SemaphoreType.DMA,
        pltpu.SemaphoreType.DMA,
    ],
    compiler_params=pltpu.CompilerParams(use_tc_tiling_on_sc=False),
)(_sc_body)


def kernel(standard_sample, edge_index, mean_param, diag_param, post_diag_param,
           alpha1, alpha2, gamma):
    f32 = jnp.float32
    zq = jnp.pad(standard_sample.astype(f32),
                 ((0, W - N_SAMPLES), (0, N_PAD - N_NODES)))
    dg = jnp.pad(diag_param, (0, N_PAD - N_NODES)).reshape(1, N_PAD)
    pd = jnp.pad(post_diag_param, (0, N_PAD - N_NODES)).reshape(1, N_PAD)
    mn = jnp.pad(mean_param, (0, N_PAD - N_NODES)).reshape(1, N_PAD)
    zeros_hbm = jnp.zeros((ZROWS, W), f32)
    scalars = jnp.stack([alpha1, alpha2, gamma]).astype(f32).reshape(1, 3)

    grid = (N_PAD // NBC,)
    blk_spec = pl.BlockSpec((W, NBC), lambda i: (0, i))
    par_spec = pl.BlockSpec((1, NBC), lambda i: (0, i))

    xs_packed = pl.pallas_call(
        _prep_body,
        grid=grid,
        in_specs=[blk_spec, par_spec, par_spec, par_spec],
        out_specs=blk_spec,
        out_shape=jax.ShapeDtypeStruct((W, N_PAD), f32),
    )(zq, dg, pd, mn)
    xs_rows = xs_packed.T

    acc = _sc_call(xs_rows, edge_index, zeros_hbm)

    out_packed = pl.pallas_call(
        _final_body,
        grid=grid,
        in_specs=[pl.BlockSpec(memory_space=pltpu.SMEM),
                  blk_spec,
                  pl.BlockSpec((NC, NBC, W), lambda i: (0, i, 0))],
        out_specs=blk_spec,
        out_shape=jax.ShapeDtypeStruct((W, N_PAD), f32),
    )(scalars, xs_packed, acc)

    return out_packed[:N_SAMPLES, :N_NODES]
